# baseline (speedup 1.0000x reference)
import functools

import jax
import jax.numpy as jnp
from jax import lax
from jax.experimental import pallas as pl
from jax.experimental.pallas import tpu as pltpu
from jax.experimental.pallas import tpu_sc as plsc

N_TOK = 9216
N_CODES = 8192
D = 256

BZ = 512
BE = 1024

def _argmin_body(z_ref, e_ref, idx_ref, minv_ref, mini_ref):
    j = pl.program_id(1)
    nj = pl.num_programs(1)
    z = z_ref[...]
    e = e_ref[...]
    mm = lax.dot_general(z, e, (((1,), (1,)), ((), ())),
                         preferred_element_type=jnp.float32)
    zn = jnp.sum(z * z, axis=1, keepdims=True)
    en = jnp.sum(e * e, axis=1, keepdims=True)
    d = (zn + en.T) - 2.0 * mm
    lv = jnp.min(d, axis=1, keepdims=True)
    cols = lax.broadcasted_iota(jnp.int32, d.shape, 1)
    li = jnp.min(jnp.where(d == lv, cols, 2 ** 30), axis=1, keepdims=True)
    li = li + j * BE

    @pl.when(j == 0)
    def _():
        minv_ref[...] = lv
        mini_ref[...] = li

    @pl.when(j > 0)
    def _():
        better = lv < minv_ref[...]
        minv_ref[...] = jnp.where(better, lv, minv_ref[...])
        mini_ref[...] = jnp.where(better, li, mini_ref[...])

    @pl.when(j == nj - 1)
    def _():
        idx_ref[...] = mini_ref[...]


def _argmin_call(z, embedding):
    grid = (N_TOK // BZ, N_CODES // BE)
    return pl.pallas_call(
        _argmin_body,
        grid=grid,
        in_specs=[
            pl.BlockSpec((BZ, D), lambda i, j: (i, 0)),
            pl.BlockSpec((BE, D), lambda i, j: (j, 0)),
        ],
        out_specs=pl.BlockSpec((BZ, 1), lambda i, j: (i, 0)),
        out_shape=jax.ShapeDtypeStruct((N_TOK, 1), jnp.int32),
        scratch_shapes=[
            pltpu.VMEM((BZ, 1), jnp.float32),
            pltpu.VMEM((BZ, 1), jnp.int32),
        ],
        compiler_params=pltpu.CompilerParams(
            dimension_semantics=("parallel", "arbitrary"),
        ),
    )(z, embedding)


_NW = 32
_BPW = N_TOK // _NW


def _gather_call(embedding, idx):
    mesh = plsc.VectorSubcoreMesh(core_axis_name="c", subcore_axis_name="s")

    @functools.partial(
        pl.kernel,
        mesh=mesh,
        out_type=jax.ShapeDtypeStruct((N_TOK, D), jnp.float32),
        scratch_types=[
            pltpu.VMEM((_BPW,), jnp.int32),
            pltpu.VMEM((_BPW, D), jnp.float32),
            pltpu.SemaphoreType.DMA,
        ],
    )
    def k(table_hbm, idx_hbm, out_hbm, idx_v, rows_v, sem):
        wid = lax.axis_index("s") * 2 + lax.axis_index("c")
        base = wid * _BPW
        pltpu.sync_copy(idx_hbm.at[pl.ds(base, _BPW)], idx_v)
        pltpu.async_copy(table_hbm.at[idx_v], rows_v, sem).wait()
        pltpu.sync_copy(rows_v, out_hbm.at[pl.ds(base, _BPW)])

    return k(embedding, idx)


def kernel(z, embedding):
    idx = _argmin_call(z, embedding).reshape(N_TOK)
    z_q = _gather_call(embedding, idx)
    return z_q, idx


# speedup vs baseline: 1.2118x; 1.2118x over previous
import functools

import jax
import jax.numpy as jnp
from jax import lax
from jax.experimental import pallas as pl
from jax.experimental.pallas import tpu as pltpu
from jax.experimental.pallas import tpu_sc as plsc

N_TOK = 9216
N_CODES = 8192
D = 256

BZ = 1024
BE = 1024


def _argmin_body(z2_ref, et_ref, idx_ref, minv_ref, mini_ref, zn_ref, en_ref):
    i = pl.program_id(0)
    j = pl.program_id(1)
    nj = pl.num_programs(1)
    z2 = z2_ref[...]
    et = et_ref[...]
    mm2 = lax.dot_general(z2, et, (((1,), (0,)), ((), ())),
                          preferred_element_type=jnp.float32)

    @pl.when(j == 0)
    def _():
        zn_ref[...] = 0.25 * jnp.sum(z2 * z2, axis=1, keepdims=True)

    @pl.when(i == 0)
    def _():
        en_ref[:, pl.ds(j * BE, BE)] = jnp.sum(et * et, axis=0, keepdims=True)

    d = (zn_ref[...] + en_ref[:, pl.ds(j * BE, BE)]) - mm2
    lv = jnp.min(d, axis=1, keepdims=True)
    colsf = lax.broadcasted_iota(jnp.int32, d.shape, 1).astype(jnp.float32)
    lif = jnp.min(jnp.where(d == lv, colsf, 3e38), axis=1, keepdims=True)
    li = lif.astype(jnp.int32) + j * BE

    @pl.when(j == 0)
    def _():
        minv_ref[...] = lv
        mini_ref[...] = li

    @pl.when(j > 0)
    def _():
        better = lv < minv_ref[...]
        minv_ref[...] = jnp.where(better, lv, minv_ref[...])
        mini_ref[...] = jnp.where(better, li, mini_ref[...])

    @pl.when(j == nj - 1)
    def _():
        idx_ref[...] = mini_ref[...]


def _argmin_call(z2, emb_t):
    grid = (N_TOK // BZ, N_CODES // BE)
    return pl.pallas_call(
        _argmin_body,
        grid=grid,
        in_specs=[
            pl.BlockSpec((BZ, D), lambda i, j: (i, 0)),
            pl.BlockSpec((D, BE), lambda i, j: (0, j)),
        ],
        out_specs=pl.BlockSpec((BZ, 1), lambda i, j: (i, 0)),
        out_shape=jax.ShapeDtypeStruct((N_TOK, 1), jnp.int32),
        scratch_shapes=[
            pltpu.VMEM((BZ, 1), jnp.float32),
            pltpu.VMEM((BZ, 1), jnp.int32),
            pltpu.VMEM((BZ, 1), jnp.float32),
            pltpu.VMEM((1, N_CODES), jnp.float32),
        ],
        compiler_params=pltpu.CompilerParams(
            dimension_semantics=("parallel", "arbitrary"),
        ),
    )(z2, emb_t)


_NW = 32
_BPW = N_TOK // _NW


def _gather_call(embedding, idx):
    mesh = plsc.VectorSubcoreMesh(core_axis_name="c", subcore_axis_name="s")

    @functools.partial(
        pl.kernel,
        mesh=mesh,
        out_type=jax.ShapeDtypeStruct((N_TOK, D), jnp.float32),
        scratch_types=[
            pltpu.VMEM((_BPW,), jnp.int32),
            pltpu.VMEM((_BPW, D), jnp.float32),
            pltpu.SemaphoreType.DMA,
        ],
    )
    def k(table_hbm, idx_hbm, out_hbm, idx_v, rows_v, sem):
        wid = lax.axis_index("s") * 2 + lax.axis_index("c")
        base = wid * _BPW
        pltpu.sync_copy(idx_hbm.at[pl.ds(base, _BPW)], idx_v)
        pltpu.async_copy(table_hbm.at[idx_v], rows_v, sem).wait()
        pltpu.sync_copy(rows_v, out_hbm.at[pl.ds(base, _BPW)])

    return k(embedding, idx)


def kernel(z, embedding):
    z2 = z + z
    emb_t = embedding.T
    idx = _argmin_call(z2, emb_t).reshape(N_TOK)
    z_q = _gather_call(embedding, idx)
    return z_q, idx
